# Initial kernel scaffold; baseline (speedup 1.0000x reference)
import jax
import jax.numpy as jnp
from jax.experimental import pallas as pl


def kernel(feat_p3, feat_p4, feat_p5, feat_p6, conv_w, conv_b, cls_w, cls_b, bbox_w, bbox_b):
    raise NotImplementedError("write your pallas kernel here")


# speedup vs baseline: 1.2072x; 1.2072x over previous
import functools

import jax
import jax.numpy as jnp
from jax.experimental import pallas as pl

_C = 256
_NH = 16
_MM_DTYPE = jnp.bfloat16


def _rpn_body(x_ref, w_ref, cb_ref, hw_ref, hb_ref, o_ref, *, PW, BM, NB):
    def band(b, _):
        m0 = b * BM
        xs = [x_ref[0, pl.ds(m0 + dy * PW, BM + 8), :] for dy in range(3)]
        accs = []
        for dx in range(3):
            acc = jnp.zeros((BM + 8, _C), jnp.float32)
            for dy in range(3):
                acc += jnp.dot(xs[dy], w_ref[dy * 3 + dx],
                               preferred_element_type=jnp.float32)
            accs.append(acc)
        conv = accs[0][0:BM] + accs[1][1:BM + 1] + accs[2][2:BM + 2]
        t = jnp.maximum(conv + cb_ref[0, :][None, :], 0.0)
        h = jnp.dot(t.astype(_MM_DTYPE), hw_ref[...],
                    preferred_element_type=jnp.float32) + hb_ref[0, :][None, :]
        o_ref[0, pl.ds(m0, BM), :] = h
        return 0

    jax.lax.fori_loop(0, NB, band, 0, unroll=False)


@functools.partial(jax.jit, static_argnums=(5, 6))
def _rpn_level(x, w9, cb, hw, hb, S, BM):
    N = x.shape[0]
    PW = -(-(S + 2) // 8) * 8
    R = (S + 3) * PW
    NB = (S * PW) // BM
    xt = jnp.transpose(x, (0, 2, 3, 1))
    xp = jnp.pad(xt, ((0, 0), (1, 2), (1, PW - S - 1), (0, 0)))
    xf = xp.reshape(N, R, _C).astype(_MM_DTYPE)

    body = functools.partial(_rpn_body, PW=PW, BM=BM, NB=NB)
    o = pl.pallas_call(
        body,
        grid=(N,),
        in_specs=[
            pl.BlockSpec((1, R, _C), lambda n: (n, 0, 0)),
            pl.BlockSpec((9, _C, _C), lambda n: (0, 0, 0)),
            pl.BlockSpec((1, _C), lambda n: (0, 0)),
            pl.BlockSpec((_C, _NH), lambda n: (0, 0)),
            pl.BlockSpec((1, _NH), lambda n: (0, 0)),
        ],
        out_specs=pl.BlockSpec((1, S * PW, _NH), lambda n: (n, 0, 0)),
        out_shape=jax.ShapeDtypeStruct((N, S * PW, _NH), jnp.float32),
    )(xf, w9, cb, hw, hb)

    o = o.reshape(N, S, PW, _NH)[:, :, :S, :15]
    o = jnp.transpose(o, (0, 3, 1, 2))
    return o[:, :3], o[:, 3:]


def kernel(feat_p3, feat_p4, feat_p5, feat_p6,
           conv_w, conv_b, cls_w, cls_b, bbox_w, bbox_b):
    w9 = jnp.transpose(conv_w, (2, 3, 1, 0)).reshape(9, _C, _C)
    w9 = w9.astype(_MM_DTYPE)
    cb = conv_b.reshape(1, _C)
    hw = jnp.concatenate([cls_w[:, :, 0, 0], bbox_w[:, :, 0, 0]], axis=0)
    hw = jnp.pad(hw, ((0, 1), (0, 0))).T.astype(_MM_DTYPE)
    hb = jnp.pad(jnp.concatenate([cls_b, bbox_b]), (0, 1)).reshape(1, _NH)

    cls_out, bbox_out = [], []
    for x, S, BM in ((feat_p3, 128, 1088), (feat_p4, 64, 1152),
                     (feat_p5, 32, 1280), (feat_p6, 16, 384)):
        c, b = _rpn_level(x, w9, cb, hw, hb, S, BM)
        cls_out.append(c)
        bbox_out.append(b)
    return tuple(cls_out) + tuple(bbox_out)


# speedup vs baseline: 1.2459x; 1.0320x over previous
import functools

import jax
import jax.numpy as jnp
from jax.experimental import pallas as pl

_C = 256
_NH = 16
_MM_DTYPE = jnp.bfloat16


def _rpn_body(x_ref, w_ref, cb_ref, hw_ref, hb_ref, o_ref, *, PW, BM, NB):
    def band(b, _):
        m0 = b * BM
        xs3 = jnp.concatenate(
            [x_ref[0, pl.ds(m0 + dy * PW, BM + 8), :] for dy in range(3)],
            axis=1)
        accs = [jnp.dot(xs3, w_ref[dx], preferred_element_type=jnp.float32)
                for dx in range(3)]
        conv = accs[0][0:BM] + accs[1][1:BM + 1] + accs[2][2:BM + 2]
        t = jnp.maximum(conv + cb_ref[0, :][None, :], 0.0)
        h = jnp.dot(t.astype(_MM_DTYPE), hw_ref[...],
                    preferred_element_type=jnp.float32) + hb_ref[0, :][None, :]
        o_ref[0, pl.ds(m0, BM), :] = h
        return 0

    jax.lax.fori_loop(0, NB, band, 0, unroll=False)


@functools.partial(jax.jit, static_argnums=(5, 6))
def _rpn_level(x, w9, cb, hw, hb, S, BM):
    N = x.shape[0]
    PW = -(-(S + 2) // 8) * 8
    R = (S + 3) * PW
    NB = (S * PW) // BM
    xt = jnp.transpose(x, (0, 2, 3, 1))
    xp = jnp.pad(xt, ((0, 0), (1, 2), (1, PW - S - 1), (0, 0)))
    xf = xp.reshape(N, R, _C).astype(_MM_DTYPE)

    body = functools.partial(_rpn_body, PW=PW, BM=BM, NB=NB)
    o = pl.pallas_call(
        body,
        grid=(N,),
        in_specs=[
            pl.BlockSpec((1, R, _C), lambda n: (n, 0, 0)),
            pl.BlockSpec((3, 3 * _C, _C), lambda n: (0, 0, 0)),
            pl.BlockSpec((1, _C), lambda n: (0, 0)),
            pl.BlockSpec((_C, _NH), lambda n: (0, 0)),
            pl.BlockSpec((1, _NH), lambda n: (0, 0)),
        ],
        out_specs=pl.BlockSpec((1, S * PW, _NH), lambda n: (n, 0, 0)),
        out_shape=jax.ShapeDtypeStruct((N, S * PW, _NH), jnp.float32),
    )(xf, w9, cb, hw, hb)

    o = o.reshape(N, S, PW, _NH)[:, :, :S, :15]
    o = jnp.transpose(o, (0, 3, 1, 2))
    return o[:, :3], o[:, 3:]


def kernel(feat_p3, feat_p4, feat_p5, feat_p6,
           conv_w, conv_b, cls_w, cls_b, bbox_w, bbox_b):
    w9 = jnp.transpose(conv_w, (3, 2, 1, 0)).reshape(3, 3 * _C, _C)
    w9 = w9.astype(_MM_DTYPE)
    cb = conv_b.reshape(1, _C)
    hw = jnp.concatenate([cls_w[:, :, 0, 0], bbox_w[:, :, 0, 0]], axis=0)
    hw = jnp.pad(hw, ((0, 1), (0, 0))).T.astype(_MM_DTYPE)
    hb = jnp.pad(jnp.concatenate([cls_b, bbox_b]), (0, 1)).reshape(1, _NH)

    cls_out, bbox_out = [], []
    for x, S, BM in ((feat_p3, 128, 2176), (feat_p4, 64, 2304),
                     (feat_p5, 32, 1280), (feat_p6, 16, 384)):
        c, b = _rpn_level(x, w9, cb, hw, hb, S, BM)
        cls_out.append(c)
        bbox_out.append(b)
    return tuple(cls_out) + tuple(bbox_out)


# speedup vs baseline: 1.6181x; 1.2988x over previous
import functools

import jax
import jax.numpy as jnp
from jax.experimental import pallas as pl
from jax.experimental.pallas import tpu as pltpu

_C = 256
_NH = 16
_MM_DTYPE = jnp.bfloat16


def _rpn_body(x_ref, w_ref, cb_ref, hw_ref, hb_ref, o_ref, xs_ref,
              *, S, BM, CH):
    IMG0 = S + 8
    SS = S * S

    xs_ref[0:IMG0, :] = jnp.zeros((IMG0, _C), _MM_DTYPE)
    xs_ref[IMG0 + SS:IMG0 + SS + S + 8, :] = jnp.zeros((S + 8, _C), _MM_DTYPE)
    for c in range(SS // CH):
        v = x_ref[0, :, c * CH:(c + 1) * CH]
        xs_ref[IMG0 + c * CH:IMG0 + (c + 1) * CH, :] = (
            jnp.transpose(v).astype(_MM_DTYPE))

    def band(b, _):
        m0 = b * BM
        xs3 = jnp.concatenate(
            [xs_ref[pl.ds(m0 + dy * S, BM + 16), :] for dy in range(3)],
            axis=1)
        accs = [jnp.dot(xs3, w_ref[dx], preferred_element_type=jnp.float32)
                for dx in range(3)]
        col = (jax.lax.broadcasted_iota(jnp.int32, (BM, 1), 0) + m0) & (S - 1)
        a0 = jnp.where(col != 0, accs[0][7:BM + 7], 0.0)
        a2 = jnp.where(col != S - 1, accs[2][9:BM + 9], 0.0)
        conv = a0 + accs[1][8:BM + 8] + a2
        t = jnp.maximum(conv + cb_ref[0, :][None, :], 0.0)
        h = jnp.dot(t.astype(_MM_DTYPE), hw_ref[...],
                    preferred_element_type=jnp.float32) + hb_ref[0, :][None, :]
        o_ref[0, :, pl.ds(m0, BM)] = jnp.transpose(h)
        return 0

    jax.lax.fori_loop(0, SS // BM, band, 0, unroll=False)


@functools.partial(jax.jit, static_argnums=(5, 6, 7))
def _rpn_level(x, w3, cb, hw, hb, S, BM, CH):
    N = x.shape[0]
    SS = S * S
    xr = x.reshape(N, _C, SS)

    body = functools.partial(_rpn_body, S=S, BM=BM, CH=CH)
    o = pl.pallas_call(
        body,
        grid=(N,),
        in_specs=[
            pl.BlockSpec((1, _C, SS), lambda n: (n, 0, 0)),
            pl.BlockSpec((3, 3 * _C, _C), lambda n: (0, 0, 0)),
            pl.BlockSpec((1, _C), lambda n: (0, 0)),
            pl.BlockSpec((_C, _NH), lambda n: (0, 0)),
            pl.BlockSpec((1, _NH), lambda n: (0, 0)),
        ],
        out_specs=pl.BlockSpec((1, _NH, SS), lambda n: (n, 0, 0)),
        out_shape=jax.ShapeDtypeStruct((N, _NH, SS), jnp.float32),
        scratch_shapes=[pltpu.VMEM((SS + 2 * S + 16, _C), _MM_DTYPE)],
    )(xr, w3, cb, hw, hb)

    o = o.reshape(N, _NH, S, S)
    return o[:, :3], o[:, 3:15]


def kernel(feat_p3, feat_p4, feat_p5, feat_p6,
           conv_w, conv_b, cls_w, cls_b, bbox_w, bbox_b):
    w3 = jnp.transpose(conv_w, (3, 2, 1, 0)).reshape(3, 3 * _C, _C)
    w3 = w3.astype(_MM_DTYPE)
    cb = conv_b.reshape(1, _C)
    hw = jnp.concatenate([cls_w[:, :, 0, 0], bbox_w[:, :, 0, 0]], axis=0)
    hw = jnp.pad(hw, ((0, 1), (0, 0))).T.astype(_MM_DTYPE)
    hb = jnp.pad(jnp.concatenate([cls_b, bbox_b]), (0, 1)).reshape(1, _NH)

    cls_out, bbox_out = [], []
    for x, S, BM, CH in ((feat_p3, 128, 2048, 2048),
                         (feat_p4, 64, 2048, 2048),
                         (feat_p5, 32, 1024, 1024),
                         (feat_p6, 16, 256, 256)):
        c, b = _rpn_level(x, w3, cb, hw, hb, S, BM, CH)
        cls_out.append(c)
        bbox_out.append(b)
    return tuple(cls_out) + tuple(bbox_out)
